# baseline (speedup 1.0000x reference)
import functools

import jax
import jax.numpy as jnp
from jax import lax
from jax.experimental import pallas as pl
from jax.experimental.pallas import tpu as pltpu
from jax.experimental.pallas import tpu_sc as plsc

V = 512
E = 128
P = 8
BT = 128
S = 256
NC, NS, L = 2, 16, 16
NW = NC * NS
BT_PER_W = BT // NW

_mesh = plsc.VectorSubcoreMesh(core_axis_name="c", subcore_axis_name="s")


@functools.partial(
    pl.kernel,
    mesh=_mesh,
    compiler_params=pltpu.CompilerParams(needs_layout_passes=False),
    out_type=jax.ShapeDtypeStruct((BT, S * E), jnp.float32),
    scratch_types=[
        pltpu.VMEM((V * E,), jnp.float32),
        pltpu.VMEM((P * S,), jnp.int32),
        pltpu.VMEM((S * E,), jnp.float32),
    ],
)
def _embed_sc(x_hbm, w_hbm, out_hbm, w_v, x_v, o_v):
    wid = lax.axis_index("s") * NC + lax.axis_index("c")
    pltpu.sync_copy(w_hbm, w_v)
    iota = lax.iota(jnp.int32, L)
    lane = [jnp.full((L,), j, jnp.int32) for j in range(L)]

    def bt_body(i, carry):
        bt = wid * BT_PER_W + i
        pltpu.sync_copy(x_hbm.at[bt], x_v)

        @plsc.parallel_loop(0, S // L, step=1, unroll=1)
        def g_body(g):
            s0 = g * L
            bases = [x_v[pl.ds(p * S + s0, L)] * E for p in range(P)]
            for j in range(L):
                bv = [bases[p].at[lane[j]].get(mode="promise_in_bounds")
                      + iota for p in range(P)]
                sb = (s0 + j) * E
                for dc in range(E // L):
                    o = dc * L
                    v0 = (plsc.load_gather(w_v, [bv[0] + o])
                          + plsc.load_gather(w_v, [bv[1] + o]))
                    v1 = (plsc.load_gather(w_v, [bv[2] + o])
                          + plsc.load_gather(w_v, [bv[3] + o]))
                    v2 = (plsc.load_gather(w_v, [bv[4] + o])
                          + plsc.load_gather(w_v, [bv[5] + o]))
                    v3 = (plsc.load_gather(w_v, [bv[6] + o])
                          + plsc.load_gather(w_v, [bv[7] + o]))
                    o_v[pl.ds(sb + o, L)] = (v0 + v1) + (v2 + v3)

        pltpu.sync_copy(o_v, out_hbm.at[bt])
        return carry

    lax.fori_loop(0, BT_PER_W, bt_body, 0)


def kernel(x, W):
    xt = x.astype(jnp.int32).reshape(BT, P * S)
    wf = W.reshape(V * E)
    out = _embed_sc(xt, wf)
    out = out.reshape(16, 8, 16, 16, E)
    return jnp.transpose(out, (0, 1, 4, 2, 3))


# speedup vs baseline: 2.2395x; 2.2395x over previous
import functools

import jax
import jax.numpy as jnp
from jax import lax
from jax.experimental import pallas as pl
from jax.experimental.pallas import tpu as pltpu
from jax.experimental.pallas import tpu_sc as plsc

V = 512
E = 128
P = 8
BT = 128
S = 256
NC, NS, L = 2, 16, 16
L2 = 2 * L
EW = E // 2
NW = NC * NS
BT_PER_W = BT // NW

_mesh = plsc.VectorSubcoreMesh(core_axis_name="c", subcore_axis_name="s")


@functools.partial(
    pl.kernel,
    mesh=_mesh,
    compiler_params=pltpu.CompilerParams(needs_layout_passes=False),
    out_type=jax.ShapeDtypeStruct((BT, S * E), jnp.float32),
    scratch_types=[
        pltpu.VMEM((V * EW,), jnp.int32),
        pltpu.VMEM((P * S,), jnp.int32),
        pltpu.VMEM((S * E,), jnp.float32),
    ],
)
def _embed_sc(x_hbm, w_hbm, out_hbm, w_v, x_v, o_v):
    wid = lax.axis_index("s") * NC + lax.axis_index("c")
    pltpu.sync_copy(w_hbm, w_v)

    def bt_body(i, carry):
        bt = wid * BT_PER_W + i
        pltpu.sync_copy(x_hbm.at[bt], x_v)

        @plsc.parallel_loop(0, S // L, step=1, unroll=1)
        def g_body(g):
            s0 = g * L
            bases = [x_v[pl.ds(p * S + s0, L)] * EW for p in range(P)]
            for j in range(L):
                rows = [bases[p][j] for p in range(P)]
                sb = (s0 + j) * E
                for q in range(E // L2):
                    o = q * L
                    w = [w_v[pl.ds(rows[p] + o, L)] for p in range(P)]
                    lo0 = (plsc.bitcast(w[0] << 16, jnp.float32)
                           + plsc.bitcast(w[1] << 16, jnp.float32))
                    lo1 = (plsc.bitcast(w[2] << 16, jnp.float32)
                           + plsc.bitcast(w[3] << 16, jnp.float32))
                    lo2 = (plsc.bitcast(w[4] << 16, jnp.float32)
                           + plsc.bitcast(w[5] << 16, jnp.float32))
                    lo3 = (plsc.bitcast(w[6] << 16, jnp.float32)
                           + plsc.bitcast(w[7] << 16, jnp.float32))
                    hi0 = (plsc.bitcast(w[0], jnp.float32)
                           + plsc.bitcast(w[1], jnp.float32))
                    hi1 = (plsc.bitcast(w[2], jnp.float32)
                           + plsc.bitcast(w[3], jnp.float32))
                    hi2 = (plsc.bitcast(w[4], jnp.float32)
                           + plsc.bitcast(w[5], jnp.float32))
                    hi3 = (plsc.bitcast(w[6], jnp.float32)
                           + plsc.bitcast(w[7], jnp.float32))
                    o_v[pl.ds(sb + q * L2, L)] = (lo0 + lo1) + (lo2 + lo3)
                    o_v[pl.ds(sb + q * L2 + L, L)] = (hi0 + hi1) + (hi2 + hi3)

        pltpu.sync_copy(o_v, out_hbm.at[bt])
        return carry

    lax.fori_loop(0, BT_PER_W, bt_body, 0)


def kernel(x, W):
    xt = x.astype(jnp.int32).reshape(BT, P * S)
    wb = W.astype(jnp.bfloat16).reshape(V, E // L2, 2, L)
    wi = jnp.transpose(wb, (0, 1, 3, 2))
    wf = jax.lax.bitcast_convert_type(wi, jnp.int32).reshape(V * EW)
    out = _embed_sc(xt, wf)
    out = out.reshape(16, 8, 16, 16, E)
    return jnp.transpose(out, (0, 1, 4, 2, 3))
